# baseline (speedup 1.0000x reference)
import functools

import jax
import jax.numpy as jnp
from jax import lax
from jax.experimental import pallas as pl
from jax.experimental.pallas import tpu as pltpu
from jax.experimental.pallas import tpu_sc as plsc

N = 10000
E = 320000
D_IN = 128
HID = 128
HID4 = 32
EMB = 128
NEG_SLOPE = 0.01
EPS = 1e-5

NC = 2
NS = 16
NW = NC * NS
NPAD = 10240
EPW = E // NW
K = 80
NCHUNK = EPW // K
NB = 5
ROWS_PT = NPAD // NS


def _leaky(x):
    return jnp.where(x >= 0, x, NEG_SLOPE * x)


_IDXCH = 2000


def _degrees(edge_flat):
    mesh = plsc.VectorSubcoreMesh(core_axis_name="c", subcore_axis_name="s")

    @functools.partial(
        pl.kernel,
        mesh=mesh,
        out_type=jax.ShapeDtypeStruct((NW, NPAD), jnp.float32),
        scratch_types=[
            pltpu.VMEM((_IDXCH,), jnp.int32),
            pltpu.VMEM((NPAD,), jnp.float32),
        ],
        compiler_params=pltpu.CompilerParams(needs_layout_passes=False, use_tc_tiling_on_sc=False),
    )
    def k(edge_hbm, out_hbm, idx_v, hist_v):
        cid = lax.axis_index("c")
        sid = lax.axis_index("s")
        w = cid * NS + sid

        def zbody(i, _):
            hist_v[pl.ds(i * 16, 16)] = jnp.zeros((16,), jnp.float32)
            return 0

        lax.fori_loop(0, NPAD // 16, zbody, 0)

        per_sub = E // NS
        base = cid * E + sid * per_sub
        ones = jnp.full((16,), 1.0, jnp.float32)

        def chunk(cn, _):
            pltpu.sync_copy(edge_hbm.at[pl.ds(base + cn * _IDXCH, _IDXCH)],
                            idx_v)

            def ibody(t5, _):
                for u in range(5):
                    iv = idx_v[pl.ds((t5 * 5 + u) * 16, 16)]
                    plsc.addupdate_scatter(hist_v, [iv], ones)
                return 0

            lax.fori_loop(0, _IDXCH // 80, ibody, 0)
            return 0

        lax.fori_loop(0, per_sub // _IDXCH, chunk, 0)
        pltpu.sync_copy(hist_v, out_hbm.at[w])

    return k(edge_flat)


def _tc_pre(x, degp, W1):
    def body(x_ref, degp_ref, w1_ref, y1a_ref, y1b_ref, s_ref, r0_ref):
        xv = x_ref[...]
        y1 = jnp.dot(xv, w1_ref[...], preferred_element_type=jnp.float32)
        y1a_ref[...] = y1[:, :HID // 2]
        y1b_ref[...] = y1[:, HID // 2:]
        r0_ref[...] = jnp.mean(xv, axis=0, keepdims=True)
        dp = degp_ref[...]
        dout = jnp.sum(dp[:NS], axis=0, keepdims=True)
        din = jnp.sum(dp[NS:], axis=0, keepdims=True)
        deg = jnp.concatenate([dout, din], axis=0)
        s_ref[...] = lax.rsqrt(jnp.maximum(deg, 1.0))

    return pl.pallas_call(
        body,
        out_shape=(
            jax.ShapeDtypeStruct((N, HID // 2), jnp.float32),
            jax.ShapeDtypeStruct((N, HID // 2), jnp.float32),
            jax.ShapeDtypeStruct((2, NPAD), jnp.float32),
            jax.ShapeDtypeStruct((1, D_IN), jnp.float32),
        ),
    )(x, degp, W1)


def _ewp_pass(src1d, dst1d, ew, s):
    mesh = plsc.VectorSubcoreMesh(core_axis_name="c", subcore_axis_name="s")

    @functools.partial(
        pl.kernel,
        mesh=mesh,
        out_type=jax.ShapeDtypeStruct((E,), jnp.float32),
        scratch_types=[
            pltpu.VMEM((EPW,), jnp.int32),
            pltpu.VMEM((EPW,), jnp.int32),
            pltpu.VMEM((EPW,), jnp.float32),
            pltpu.VMEM((NPAD,), jnp.float32),
            pltpu.VMEM((NPAD,), jnp.float32),
        ],
        compiler_params=pltpu.CompilerParams(needs_layout_passes=False, use_tc_tiling_on_sc=False),
    )
    def k(src_hbm, dst_hbm, ew_hbm, s_hbm, out_hbm,
          src_v, dst_v, ewp_v, sout_v, sin_v):
        cid = lax.axis_index("c")
        sid = lax.axis_index("s")
        w = cid * NS + sid
        pltpu.sync_copy(src_hbm.at[pl.ds(w * EPW, EPW)], src_v)
        pltpu.sync_copy(dst_hbm.at[pl.ds(w * EPW, EPW)], dst_v)
        pltpu.sync_copy(ew_hbm.at[pl.ds(w * EPW, EPW)], ewp_v)
        pltpu.sync_copy(s_hbm.at[0], sout_v)
        pltpu.sync_copy(s_hbm.at[1], sin_v)

        def wbody(t5, _):
            for u in range(5):
                sl = pl.ds((t5 * 5 + u) * 16, 16)
                so = plsc.load_gather(sout_v, [src_v[sl]])
                si = plsc.load_gather(sin_v, [dst_v[sl]])
                ewp_v[sl] = ewp_v[sl] * so * si
            return 0

        lax.fori_loop(0, EPW // 80, wbody, 0)
        pltpu.sync_copy(ewp_v, out_hbm.at[pl.ds(w * EPW, EPW)])

    return k(src1d, dst1d, ew, s)


def _edge_pass(table, src1d, dst1d, ewp, dw):
    mesh = plsc.VectorSubcoreMesh(core_axis_name="c", subcore_axis_name="s")

    @functools.partial(
        pl.kernel,
        mesh=mesh,
        out_type=jax.ShapeDtypeStruct((NC, NPAD, dw), jnp.float32),
        scratch_types=[
            pltpu.VMEM((EPW,), jnp.int32),
            pltpu.VMEM((EPW,), jnp.int32),
            [pltpu.VMEM((K,), jnp.int32)] * NB,
            pltpu.VMEM((EPW,), jnp.float32),
            pltpu.VMEM((NB, K, dw), jnp.float32),
            pltpu.VMEM((NB, K, dw), jnp.float32),
            pltpu.VMEM((32, dw), jnp.float32),
            pltpu.VMEM_SHARED((NPAD, dw), jnp.float32),
            pltpu.SemaphoreType.DMA((NB,)),
            pltpu.SemaphoreType.DMA((NB,)),
        ],
        compiler_params=pltpu.CompilerParams(needs_layout_passes=False, use_tc_tiling_on_sc=False),
    )
    def k(table_hbm, src_hbm, dst_hbm, ewp_hbm, out_hbm,
          src_v, dst_v, didx_v, ewp_v, rin, rout, zb_v, accum,
          gsem, ssem):
        cid = lax.axis_index("c")
        sid = lax.axis_index("s")
        w = cid * NS + sid

        pltpu.sync_copy(src_hbm.at[pl.ds(w * EPW, EPW)], src_v)
        pltpu.sync_copy(dst_hbm.at[pl.ds(w * EPW, EPW)], dst_v)
        pltpu.sync_copy(ewp_hbm.at[pl.ds(w * EPW, EPW)], ewp_v)

        def zrow(i, _):
            for j in range(dw // 16):
                zb_v[i, pl.ds(j * 16, 16)] = jnp.zeros((16,), jnp.float32)
            return 0

        lax.fori_loop(0, 32, zrow, 0)
        for t in range(ROWS_PT // 32):
            pltpu.sync_copy(zb_v, accum.at[pl.ds(sid * ROWS_PT + t * 32, 32)])
        plsc.subcore_barrier()

        def scale(c, b):
            rib = rin.at[b]
            rob = rout.at[b]

            def ebody(e8, _):
                for u in range(8):
                    e = e8 * 8 + u
                    ewb = plsc.load_gather(
                        ewp_v, [jnp.full((16,), c * K + e, jnp.int32)])
                    for j in range(dw // 16):
                        sl = pl.ds(j * 16, 16)
                        rob[e, sl] = rib[e, sl] * ewb
                return 0

            lax.fori_loop(0, K // 8, ebody, 0)

        def gidx(c):
            return src_v.at[pl.ds(c * K, K)]

        for b in range(NB):
            pltpu.async_copy(table_hbm.at[gidx(b)], rin.at[b], gsem.at[b])

        def group(g, _):
            for b in range(NB):
                c = g * NB + b
                rib = rin.at[b]
                rob = rout.at[b]
                pltpu.make_async_copy(table_hbm.at[gidx(c)], rib,
                                      gsem.at[b]).wait()

                @pl.when(g > 0)
                def _():
                    pltpu.make_async_copy(rob, accum.at[didx_v[b]],
                                          ssem.at[b]).wait()

                for q in range(K // 16):
                    didx_v[b][pl.ds(q * 16, 16)] = dst_v[pl.ds(c * K + q * 16,
                                                               16)]
                scale(c, b)

                @pl.when(c + NB < NCHUNK)
                def _():
                    pltpu.async_copy(table_hbm.at[gidx(c + NB)], rib,
                                     gsem.at[b])

                pltpu.async_copy(rob, accum.at[didx_v[b]], ssem.at[b],
                                 add=True)
            return 0

        lax.fori_loop(0, NCHUNK // NB, group, 0)
        for b in range(NB):
            pltpu.make_async_copy(rout.at[b], accum.at[didx_v[b]],
                                  ssem.at[b]).wait()

        plsc.subcore_barrier()
        sl = pl.ds(sid * ROWS_PT, ROWS_PT)
        pltpu.sync_copy(accum.at[sl], out_hbm.at[cid, sl])

    return k(table, src1d, dst1d, ewp)


def _tc_mid(pa, pb, gamma, beta, alpha, W2):
    def body(pa_ref, pb_ref, g_ref, b_ref, a_ref, w2_ref, y2_ref, r1_ref):
        rst = jnp.concatenate(
            [pa_ref[0, :N] + pa_ref[1, :N], pb_ref[0, :N] + pb_ref[1, :N]],
            axis=1)
        m = jnp.mean(rst, axis=0, keepdims=True)
        q = jnp.mean(rst * rst, axis=0, keepdims=True)
        al = a_ref[...]
        out = rst - al * m
        var = q - (2.0 - al) * al * m * m
        h = _leaky(g_ref[...] * out * lax.rsqrt(var + EPS) + b_ref[...])
        r1_ref[...] = jnp.mean(h, axis=0, keepdims=True)
        y2_ref[...] = jnp.dot(h, w2_ref[...],
                              preferred_element_type=jnp.float32)

    return pl.pallas_call(
        body,
        out_shape=(
            jax.ShapeDtypeStruct((N, HID4), jnp.float32),
            jax.ShapeDtypeStruct((1, HID), jnp.float32),
        ),
    )(pa, pb, gamma, beta, alpha, W2)


def _tc_post(p, gamma, beta, alpha, r0, r1, We):
    def body(p_ref, g_ref, b_ref, a_ref, r0_ref, r1_ref, we_ref, out_ref):
        rst = p_ref[0, :N] + p_ref[1, :N]
        m = jnp.mean(rst, axis=0, keepdims=True)
        q = jnp.mean(rst * rst, axis=0, keepdims=True)
        al = a_ref[...]
        out = rst - al * m
        var = q - (2.0 - al) * al * m * m
        h = _leaky(g_ref[...] * out * lax.rsqrt(var + EPS) + b_ref[...])
        r2 = jnp.mean(h, axis=0, keepdims=True)
        emb = (jnp.dot(r0_ref[...], we_ref[0:D_IN, :],
                       preferred_element_type=jnp.float32)
               + jnp.dot(r1_ref[...], we_ref[D_IN:D_IN + HID, :],
                         preferred_element_type=jnp.float32)
               + jnp.dot(r2, we_ref[D_IN + HID:, :],
                         preferred_element_type=jnp.float32))
        em = jnp.mean(emb, axis=1, keepdims=True)
        ev = jnp.mean((emb - em) ** 2, axis=1, keepdims=True)
        out_ref[...] = _leaky((emb - em) * lax.rsqrt(ev + EPS))

    return pl.pallas_call(
        body,
        out_shape=jax.ShapeDtypeStruct((1, EMB), jnp.float32),
    )(p, gamma, beta, alpha, r0, r1, We)


def kernel(node_feats, edge_index, edge_weights, W1, W2, We,
           gamma1, beta1, alpha1, gamma2, beta2, alpha2):
    degp = _degrees(edge_index.reshape(2 * E))
    y1a, y1b, s, r0 = _tc_pre(node_feats, degp, W1)
    src1d = edge_index[0]
    dst1d = edge_index[1]
    ewp = _ewp_pass(src1d, dst1d, edge_weights, s)
    p1a = _edge_pass(y1a, src1d, dst1d, ewp, HID // 2)
    p1b = _edge_pass(y1b, src1d, dst1d, ewp, HID // 2)
    y2, r1 = _tc_mid(p1a, p1b, gamma1.reshape(1, HID), beta1.reshape(1, HID),
                     alpha1.reshape(1, HID), W2)
    p2 = _edge_pass(y2, src1d, dst1d, ewp, HID4)
    out = _tc_post(p2, gamma2.reshape(1, HID4), beta2.reshape(1, HID4),
                   alpha2.reshape(1, HID4), r0, r1, We)
    return out


# speedup vs baseline: 1.4036x; 1.4036x over previous
import functools

import jax
import jax.numpy as jnp
from jax import lax
from jax.experimental import pallas as pl
from jax.experimental.pallas import tpu as pltpu
from jax.experimental.pallas import tpu_sc as plsc

N = 10000
E = 320000
D_IN = 128
HID = 128
HID4 = 32
EMB = 128
NEG_SLOPE = 0.01
EPS = 1e-5

NC = 2
NS = 16
NW = NC * NS
NPAD = 10240
EPW = E // NW
K = 16
NCHUNK = EPW // K
NB = 5
ROWS_PT = NPAD // NS


def _leaky(x):
    return jnp.where(x >= 0, x, NEG_SLOPE * x)


_IDXCH = 2000


def _degrees(edge_flat):
    mesh = plsc.VectorSubcoreMesh(core_axis_name="c", subcore_axis_name="s")

    @functools.partial(
        pl.kernel,
        mesh=mesh,
        out_type=jax.ShapeDtypeStruct((NW, NPAD), jnp.float32),
        scratch_types=[
            pltpu.VMEM((_IDXCH,), jnp.int32),
            pltpu.VMEM((NPAD,), jnp.float32),
        ],
        compiler_params=pltpu.CompilerParams(needs_layout_passes=False, use_tc_tiling_on_sc=False),
    )
    def k(edge_hbm, out_hbm, idx_v, hist_v):
        cid = lax.axis_index("c")
        sid = lax.axis_index("s")
        w = cid * NS + sid

        def zbody(i, _):
            hist_v[pl.ds(i * 16, 16)] = jnp.zeros((16,), jnp.float32)
            return 0

        lax.fori_loop(0, NPAD // 16, zbody, 0)

        per_sub = E // NS
        base = cid * E + sid * per_sub
        ones = jnp.full((16,), 1.0, jnp.float32)

        def chunk(cn, _):
            pltpu.sync_copy(edge_hbm.at[pl.ds(base + cn * _IDXCH, _IDXCH)],
                            idx_v)

            def ibody(t5, _):
                for u in range(5):
                    iv = idx_v[pl.ds((t5 * 5 + u) * 16, 16)]
                    plsc.addupdate_scatter(hist_v, [iv], ones)
                return 0

            lax.fori_loop(0, _IDXCH // 80, ibody, 0)
            return 0

        lax.fori_loop(0, per_sub // _IDXCH, chunk, 0)
        pltpu.sync_copy(hist_v, out_hbm.at[w])

    return k(edge_flat)


def _tc_pre(x, degp, W1):
    def body(x_ref, degp_ref, w1_ref, y1a_ref, y1b_ref, s_ref, r0_ref):
        xv = x_ref[...]
        y1 = jnp.dot(xv, w1_ref[...], preferred_element_type=jnp.float32)
        y1a_ref[...] = y1[:, :HID // 2]
        y1b_ref[...] = y1[:, HID // 2:]
        r0_ref[...] = jnp.mean(xv, axis=0, keepdims=True)
        dp = degp_ref[...]
        dout = jnp.sum(dp[:NS], axis=0, keepdims=True)
        din = jnp.sum(dp[NS:], axis=0, keepdims=True)
        deg = jnp.concatenate([dout, din], axis=0)
        s_ref[...] = lax.rsqrt(jnp.maximum(deg, 1.0))

    return pl.pallas_call(
        body,
        out_shape=(
            jax.ShapeDtypeStruct((N, HID // 2), jnp.float32),
            jax.ShapeDtypeStruct((N, HID // 2), jnp.float32),
            jax.ShapeDtypeStruct((2, NPAD), jnp.float32),
            jax.ShapeDtypeStruct((1, D_IN), jnp.float32),
        ),
    )(x, degp, W1)


def _ewp_pass(src1d, dst1d, ew, s):
    mesh = plsc.VectorSubcoreMesh(core_axis_name="c", subcore_axis_name="s")

    @functools.partial(
        pl.kernel,
        mesh=mesh,
        out_type=jax.ShapeDtypeStruct((E,), jnp.float32),
        scratch_types=[
            pltpu.VMEM((EPW,), jnp.int32),
            pltpu.VMEM((EPW,), jnp.int32),
            pltpu.VMEM((EPW,), jnp.float32),
            pltpu.VMEM((NPAD,), jnp.float32),
            pltpu.VMEM((NPAD,), jnp.float32),
        ],
        compiler_params=pltpu.CompilerParams(needs_layout_passes=False, use_tc_tiling_on_sc=False),
    )
    def k(src_hbm, dst_hbm, ew_hbm, s_hbm, out_hbm,
          src_v, dst_v, ewp_v, sout_v, sin_v):
        cid = lax.axis_index("c")
        sid = lax.axis_index("s")
        w = cid * NS + sid
        pltpu.sync_copy(src_hbm.at[pl.ds(w * EPW, EPW)], src_v)
        pltpu.sync_copy(dst_hbm.at[pl.ds(w * EPW, EPW)], dst_v)
        pltpu.sync_copy(ew_hbm.at[pl.ds(w * EPW, EPW)], ewp_v)
        pltpu.sync_copy(s_hbm.at[0], sout_v)
        pltpu.sync_copy(s_hbm.at[1], sin_v)

        def wbody(t5, _):
            for u in range(5):
                sl = pl.ds((t5 * 5 + u) * 16, 16)
                so = plsc.load_gather(sout_v, [src_v[sl]])
                si = plsc.load_gather(sin_v, [dst_v[sl]])
                ewp_v[sl] = ewp_v[sl] * so * si
            return 0

        lax.fori_loop(0, EPW // 80, wbody, 0)
        pltpu.sync_copy(ewp_v, out_hbm.at[pl.ds(w * EPW, EPW)])

    return k(src1d, dst1d, ew, s)


def _edge_pass(table, src1d, dst1d, ewp, dw):
    mesh = plsc.VectorSubcoreMesh(core_axis_name="c", subcore_axis_name="s")

    @functools.partial(
        pl.kernel,
        mesh=mesh,
        out_type=jax.ShapeDtypeStruct((NC, NPAD, dw), jnp.float32),
        scratch_types=[
            pltpu.VMEM((EPW,), jnp.int32),
            pltpu.VMEM((EPW,), jnp.int32),
            [pltpu.VMEM((K,), jnp.int32)] * NB,
            pltpu.VMEM((EPW,), jnp.float32),
            pltpu.VMEM((NB, K, dw), jnp.float32),
            pltpu.VMEM((NB, K, dw), jnp.float32),
            pltpu.VMEM((32, dw), jnp.float32),
            pltpu.VMEM_SHARED((NPAD, dw), jnp.float32),
            pltpu.SemaphoreType.DMA((NB,)),
            pltpu.SemaphoreType.DMA((NB,)),
        ],
        compiler_params=pltpu.CompilerParams(needs_layout_passes=False, use_tc_tiling_on_sc=False),
    )
    def k(table_hbm, src_hbm, dst_hbm, ewp_hbm, out_hbm,
          src_v, dst_v, didx_v, ewp_v, rin, rout, zb_v, accum,
          gsem, ssem):
        cid = lax.axis_index("c")
        sid = lax.axis_index("s")
        w = cid * NS + sid

        pltpu.sync_copy(src_hbm.at[pl.ds(w * EPW, EPW)], src_v)
        pltpu.sync_copy(dst_hbm.at[pl.ds(w * EPW, EPW)], dst_v)
        pltpu.sync_copy(ewp_hbm.at[pl.ds(w * EPW, EPW)], ewp_v)

        def zrow(i, _):
            for j in range(dw // 16):
                zb_v[i, pl.ds(j * 16, 16)] = jnp.zeros((16,), jnp.float32)
            return 0

        lax.fori_loop(0, 32, zrow, 0)
        for t in range(ROWS_PT // 32):
            pltpu.sync_copy(zb_v, accum.at[pl.ds(sid * ROWS_PT + t * 32, 32)])
        plsc.subcore_barrier()

        def scale(c, b):
            rib = rin.at[b]
            rob = rout.at[b]

            def ebody(e8, _):
                for u in range(8):
                    e = e8 * 8 + u
                    ewb = plsc.load_gather(
                        ewp_v, [jnp.full((16,), c * K + e, jnp.int32)])
                    for j in range(dw // 16):
                        sl = pl.ds(j * 16, 16)
                        rob[e, sl] = rib[e, sl] * ewb
                return 0

            lax.fori_loop(0, K // 8, ebody, 0)

        def gidx(c):
            return src_v.at[pl.ds(c * K, K)]

        for b in range(NB):
            pltpu.async_copy(table_hbm.at[gidx(b)], rin.at[b], gsem.at[b])

        def group(g, _):
            for b in range(NB):
                c = g * NB + b
                rib = rin.at[b]
                rob = rout.at[b]
                pltpu.make_async_copy(table_hbm.at[gidx(c)], rib,
                                      gsem.at[b]).wait()

                @pl.when(g > 0)
                def _():
                    pltpu.make_async_copy(rob, accum.at[didx_v[b]],
                                          ssem.at[b]).wait()

                for q in range(K // 16):
                    didx_v[b][pl.ds(q * 16, 16)] = dst_v[pl.ds(c * K + q * 16,
                                                               16)]
                scale(c, b)

                @pl.when(c + NB < NCHUNK)
                def _():
                    pltpu.async_copy(table_hbm.at[gidx(c + NB)], rib,
                                     gsem.at[b])

                pltpu.async_copy(rob, accum.at[didx_v[b]], ssem.at[b],
                                 add=True)
            return 0

        lax.fori_loop(0, NCHUNK // NB, group, 0)
        for b in range(NB):
            pltpu.make_async_copy(rout.at[b], accum.at[didx_v[b]],
                                  ssem.at[b]).wait()

        plsc.subcore_barrier()
        sl = pl.ds(sid * ROWS_PT, ROWS_PT)
        pltpu.sync_copy(accum.at[sl], out_hbm.at[cid, sl])

    return k(table, src1d, dst1d, ewp)


def _tc_mid(pa, pb, gamma, beta, alpha, W2):
    def body(pa_ref, pb_ref, g_ref, b_ref, a_ref, w2_ref, y2_ref, r1_ref):
        rst = jnp.concatenate(
            [pa_ref[0, :N] + pa_ref[1, :N], pb_ref[0, :N] + pb_ref[1, :N]],
            axis=1)
        m = jnp.mean(rst, axis=0, keepdims=True)
        q = jnp.mean(rst * rst, axis=0, keepdims=True)
        al = a_ref[...]
        out = rst - al * m
        var = q - (2.0 - al) * al * m * m
        h = _leaky(g_ref[...] * out * lax.rsqrt(var + EPS) + b_ref[...])
        r1_ref[...] = jnp.mean(h, axis=0, keepdims=True)
        y2_ref[...] = jnp.dot(h, w2_ref[...],
                              preferred_element_type=jnp.float32)

    return pl.pallas_call(
        body,
        out_shape=(
            jax.ShapeDtypeStruct((N, HID4), jnp.float32),
            jax.ShapeDtypeStruct((1, HID), jnp.float32),
        ),
    )(pa, pb, gamma, beta, alpha, W2)


def _tc_post(p, gamma, beta, alpha, r0, r1, We):
    def body(p_ref, g_ref, b_ref, a_ref, r0_ref, r1_ref, we_ref, out_ref):
        rst = p_ref[0, :N] + p_ref[1, :N]
        m = jnp.mean(rst, axis=0, keepdims=True)
        q = jnp.mean(rst * rst, axis=0, keepdims=True)
        al = a_ref[...]
        out = rst - al * m
        var = q - (2.0 - al) * al * m * m
        h = _leaky(g_ref[...] * out * lax.rsqrt(var + EPS) + b_ref[...])
        r2 = jnp.mean(h, axis=0, keepdims=True)
        emb = (jnp.dot(r0_ref[...], we_ref[0:D_IN, :],
                       preferred_element_type=jnp.float32)
               + jnp.dot(r1_ref[...], we_ref[D_IN:D_IN + HID, :],
                         preferred_element_type=jnp.float32)
               + jnp.dot(r2, we_ref[D_IN + HID:, :],
                         preferred_element_type=jnp.float32))
        em = jnp.mean(emb, axis=1, keepdims=True)
        ev = jnp.mean((emb - em) ** 2, axis=1, keepdims=True)
        out_ref[...] = _leaky((emb - em) * lax.rsqrt(ev + EPS))

    return pl.pallas_call(
        body,
        out_shape=jax.ShapeDtypeStruct((1, EMB), jnp.float32),
    )(p, gamma, beta, alpha, r0, r1, We)


def kernel(node_feats, edge_index, edge_weights, W1, W2, We,
           gamma1, beta1, alpha1, gamma2, beta2, alpha2):
    degp = _degrees(edge_index.reshape(2 * E))
    y1a, y1b, s, r0 = _tc_pre(node_feats, degp, W1)
    src1d = edge_index[0]
    dst1d = edge_index[1]
    ewp = _ewp_pass(src1d, dst1d, edge_weights, s)
    p1a = _edge_pass(y1a, src1d, dst1d, ewp, HID // 2)
    p1b = _edge_pass(y1b, src1d, dst1d, ewp, HID // 2)
    y2, r1 = _tc_mid(p1a, p1b, gamma1.reshape(1, HID), beta1.reshape(1, HID),
                     alpha1.reshape(1, HID), W2)
    p2 = _edge_pass(y2, src1d, dst1d, ewp, HID4)
    out = _tc_post(p2, gamma2.reshape(1, HID4), beta2.reshape(1, HID4),
                   alpha2.reshape(1, HID4), r0, r1, We)
    return out


# speedup vs baseline: 1.6758x; 1.1939x over previous
import functools

import jax
import jax.numpy as jnp
from jax import lax
from jax.experimental import pallas as pl
from jax.experimental.pallas import tpu as pltpu
from jax.experimental.pallas import tpu_sc as plsc

N = 10000
E = 320000
D_IN = 128
HID = 128
HID4 = 32
EMB = 128
NEG_SLOPE = 0.01
EPS = 1e-5

NC = 2
NS = 16
NW = NC * NS
NPAD = 10240
EPW = E // NW
K = 16
NCHUNK = EPW // K
NB = 5
ROWS_PT = NPAD // NS


def _leaky(x):
    return jnp.where(x >= 0, x, NEG_SLOPE * x)


_IDXCH = 2000


def _degrees(edge_flat):
    mesh = plsc.VectorSubcoreMesh(core_axis_name="c", subcore_axis_name="s")

    @functools.partial(
        pl.kernel,
        mesh=mesh,
        out_type=jax.ShapeDtypeStruct((NW, NPAD), jnp.float32),
        scratch_types=[
            pltpu.VMEM((_IDXCH,), jnp.int32),
            pltpu.VMEM((NPAD,), jnp.float32),
        ],
        compiler_params=pltpu.CompilerParams(needs_layout_passes=False, use_tc_tiling_on_sc=False),
    )
    def k(edge_hbm, out_hbm, idx_v, hist_v):
        cid = lax.axis_index("c")
        sid = lax.axis_index("s")
        w = cid * NS + sid

        def zbody(i, _):
            hist_v[pl.ds(i * 16, 16)] = jnp.zeros((16,), jnp.float32)
            return 0

        lax.fori_loop(0, NPAD // 16, zbody, 0)

        per_sub = E // NS
        base = cid * E + sid * per_sub
        ones = jnp.full((16,), 1.0, jnp.float32)

        def chunk(cn, _):
            pltpu.sync_copy(edge_hbm.at[pl.ds(base + cn * _IDXCH, _IDXCH)],
                            idx_v)

            def ibody(t5, _):
                for u in range(5):
                    iv = idx_v[pl.ds((t5 * 5 + u) * 16, 16)]
                    plsc.addupdate_scatter(hist_v, [iv], ones)
                return 0

            lax.fori_loop(0, _IDXCH // 80, ibody, 0)
            return 0

        lax.fori_loop(0, per_sub // _IDXCH, chunk, 0)
        pltpu.sync_copy(hist_v, out_hbm.at[w])

    return k(edge_flat)


def _tc_pre(x, degp, W1):
    def body(x_ref, degp_ref, w1_ref, y1a_ref, y1b_ref, s_ref, r0_ref):
        xv = x_ref[...]
        y1 = jnp.dot(xv, w1_ref[...], preferred_element_type=jnp.float32)
        y1a_ref[...] = y1[:, :HID // 2]
        y1b_ref[...] = y1[:, HID // 2:]
        r0_ref[...] = jnp.mean(xv, axis=0, keepdims=True)
        dp = degp_ref[...]
        dout = jnp.sum(dp[:NS], axis=0, keepdims=True)
        din = jnp.sum(dp[NS:], axis=0, keepdims=True)
        deg = jnp.concatenate([dout, din], axis=0)
        s_ref[...] = lax.rsqrt(jnp.maximum(deg, 1.0))

    return pl.pallas_call(
        body,
        out_shape=(
            jax.ShapeDtypeStruct((N, HID // 2), jnp.float32),
            jax.ShapeDtypeStruct((N, HID // 2), jnp.float32),
            jax.ShapeDtypeStruct((2, NPAD), jnp.float32),
            jax.ShapeDtypeStruct((1, D_IN), jnp.float32),
        ),
    )(x, degp, W1)


def _ewp_pass(src1d, dst1d, ew, s):
    mesh = plsc.VectorSubcoreMesh(core_axis_name="c", subcore_axis_name="s")

    @functools.partial(
        pl.kernel,
        mesh=mesh,
        out_type=jax.ShapeDtypeStruct((E,), jnp.float32),
        scratch_types=[
            pltpu.VMEM((EPW,), jnp.int32),
            pltpu.VMEM((EPW,), jnp.int32),
            pltpu.VMEM((EPW,), jnp.float32),
            pltpu.VMEM((NPAD,), jnp.float32),
            pltpu.VMEM((NPAD,), jnp.float32),
        ],
        compiler_params=pltpu.CompilerParams(needs_layout_passes=False, use_tc_tiling_on_sc=False),
    )
    def k(src_hbm, dst_hbm, ew_hbm, s_hbm, out_hbm,
          src_v, dst_v, ewp_v, sout_v, sin_v):
        cid = lax.axis_index("c")
        sid = lax.axis_index("s")
        w = cid * NS + sid
        pltpu.sync_copy(src_hbm.at[pl.ds(w * EPW, EPW)], src_v)
        pltpu.sync_copy(dst_hbm.at[pl.ds(w * EPW, EPW)], dst_v)
        pltpu.sync_copy(ew_hbm.at[pl.ds(w * EPW, EPW)], ewp_v)
        pltpu.sync_copy(s_hbm.at[0], sout_v)
        pltpu.sync_copy(s_hbm.at[1], sin_v)

        def wbody(t5, _):
            for u in range(5):
                sl = pl.ds((t5 * 5 + u) * 16, 16)
                so = plsc.load_gather(sout_v, [src_v[sl]])
                si = plsc.load_gather(sin_v, [dst_v[sl]])
                ewp_v[sl] = ewp_v[sl] * so * si
            return 0

        lax.fori_loop(0, EPW // 80, wbody, 0)
        pltpu.sync_copy(ewp_v, out_hbm.at[pl.ds(w * EPW, EPW)])

    return k(src1d, dst1d, ew, s)


DW1 = HID // 2


def _edge_pass_dual(table_a, table_b, src1d, dst1d, ewp):
    mesh = plsc.VectorSubcoreMesh(core_axis_name="c", subcore_axis_name="s")

    @functools.partial(
        pl.kernel,
        mesh=mesh,
        out_type=(
            jax.ShapeDtypeStruct((NC, N, DW1), jnp.float32),
            jax.ShapeDtypeStruct((NC, N, DW1), jnp.float32),
        ),
        scratch_types=[
            pltpu.VMEM((EPW,), jnp.int32),
            pltpu.VMEM((EPW,), jnp.int32),
            [pltpu.VMEM((K,), jnp.int32)] * NB,
            pltpu.VMEM((EPW,), jnp.float32),
            pltpu.VMEM((NB, K, DW1), jnp.float32),
            pltpu.VMEM((NB, K, DW1), jnp.float32),
            pltpu.VMEM((NB, K, DW1), jnp.float32),
            pltpu.VMEM((NB, K, DW1), jnp.float32),
            pltpu.VMEM_SHARED((N, DW1), jnp.float32),
            pltpu.VMEM_SHARED((N, DW1), jnp.float32),
            pltpu.SemaphoreType.DMA((NB,)),
            pltpu.SemaphoreType.DMA((NB,)),
            pltpu.SemaphoreType.DMA((NB,)),
            pltpu.SemaphoreType.DMA((NB,)),
        ],
        compiler_params=pltpu.CompilerParams(needs_layout_passes=False,
                                             use_tc_tiling_on_sc=False),
    )
    def k(ta_hbm, tb_hbm, src_hbm, dst_hbm, ewp_hbm, outa_hbm, outb_hbm,
          src_v, dst_v, didx_v, ewp_v, rina, rinb, routa, routb,
          acca, accb, gsa, gsb, ssa, ssb):
        cid = lax.axis_index("c")
        sid = lax.axis_index("s")
        w = cid * NS + sid

        pltpu.sync_copy(src_hbm.at[pl.ds(w * EPW, EPW)], src_v)
        pltpu.sync_copy(dst_hbm.at[pl.ds(w * EPW, EPW)], dst_v)
        pltpu.sync_copy(ewp_hbm.at[pl.ds(w * EPW, EPW)], ewp_v)

        for i in range(K):
            for j in range(DW1 // 16):
                routa[0, i, pl.ds(j * 16, 16)] = jnp.zeros((16,), jnp.float32)

        @pl.when(sid < 10)
        def _():
            def zcopy(q, _):
                dst = pl.ds(sid * 1000 + q * 8, 8)
                pltpu.sync_copy(routa.at[0, pl.ds(0, 8)], acca.at[dst])
                pltpu.sync_copy(routa.at[0, pl.ds(0, 8)], accb.at[dst])
                return 0

            lax.fori_loop(0, 125, zcopy, 0)

        plsc.subcore_barrier()

        def scale(c, b):
            def ebody(e8, _):
                for u in range(8):
                    e = e8 * 8 + u
                    ewb = plsc.load_gather(
                        ewp_v, [jnp.full((16,), c * K + e, jnp.int32)])
                    for j in range(DW1 // 16):
                        sl = pl.ds(j * 16, 16)
                        routa[b, e, sl] = rina[b, e, sl] * ewb
                        routb[b, e, sl] = rinb[b, e, sl] * ewb
                return 0

            lax.fori_loop(0, K // 8, ebody, 0)

        def gidx(c):
            return src_v.at[pl.ds(c * K, K)]

        for b in range(NB):
            pltpu.async_copy(ta_hbm.at[gidx(b)], rina.at[b], gsa.at[b])
            pltpu.async_copy(tb_hbm.at[gidx(b)], rinb.at[b], gsb.at[b])

        def group(g, _):
            for b in range(NB):
                c = g * NB + b
                pltpu.make_async_copy(ta_hbm.at[gidx(c)], rina.at[b],
                                      gsa.at[b]).wait()
                pltpu.make_async_copy(tb_hbm.at[gidx(c)], rinb.at[b],
                                      gsb.at[b]).wait()

                @pl.when(g > 0)
                def _():
                    pltpu.make_async_copy(routa.at[b], acca.at[didx_v[b]],
                                          ssa.at[b]).wait()
                    pltpu.make_async_copy(routb.at[b], accb.at[didx_v[b]],
                                          ssb.at[b]).wait()

                for q in range(K // 16):
                    didx_v[b][pl.ds(q * 16, 16)] = dst_v[pl.ds(c * K + q * 16,
                                                               16)]
                scale(c, b)

                @pl.when(c + NB < NCHUNK)
                def _():
                    pltpu.async_copy(ta_hbm.at[gidx(c + NB)], rina.at[b],
                                     gsa.at[b])
                    pltpu.async_copy(tb_hbm.at[gidx(c + NB)], rinb.at[b],
                                     gsb.at[b])

                pltpu.async_copy(routa.at[b], acca.at[didx_v[b]], ssa.at[b],
                                 add=True)
                pltpu.async_copy(routb.at[b], accb.at[didx_v[b]], ssb.at[b],
                                 add=True)
            return 0

        lax.fori_loop(0, NCHUNK // NB, group, 0)
        for b in range(NB):
            pltpu.make_async_copy(routa.at[b], acca.at[didx_v[b]],
                                  ssa.at[b]).wait()
            pltpu.make_async_copy(routb.at[b], accb.at[didx_v[b]],
                                  ssb.at[b]).wait()

        plsc.subcore_barrier()

        @pl.when(sid < 10)
        def _():
            sl = pl.ds(sid * 1000, 1000)
            pltpu.sync_copy(acca.at[sl], outa_hbm.at[cid, sl])
            pltpu.sync_copy(accb.at[sl], outb_hbm.at[cid, sl])

    return k(table_a, table_b, src1d, dst1d, ewp)


def _edge_pass(table, src1d, dst1d, ewp, dw):
    mesh = plsc.VectorSubcoreMesh(core_axis_name="c", subcore_axis_name="s")

    @functools.partial(
        pl.kernel,
        mesh=mesh,
        out_type=jax.ShapeDtypeStruct((NC, NPAD, dw), jnp.float32),
        scratch_types=[
            pltpu.VMEM((EPW,), jnp.int32),
            pltpu.VMEM((EPW,), jnp.int32),
            [pltpu.VMEM((K,), jnp.int32)] * NB,
            pltpu.VMEM((EPW,), jnp.float32),
            pltpu.VMEM((NB, K, dw), jnp.float32),
            pltpu.VMEM((NB, K, dw), jnp.float32),
            pltpu.VMEM((32, dw), jnp.float32),
            pltpu.VMEM_SHARED((NPAD, dw), jnp.float32),
            pltpu.SemaphoreType.DMA((NB,)),
            pltpu.SemaphoreType.DMA((NB,)),
        ],
        compiler_params=pltpu.CompilerParams(needs_layout_passes=False, use_tc_tiling_on_sc=False),
    )
    def k(table_hbm, src_hbm, dst_hbm, ewp_hbm, out_hbm,
          src_v, dst_v, didx_v, ewp_v, rin, rout, zb_v, accum,
          gsem, ssem):
        cid = lax.axis_index("c")
        sid = lax.axis_index("s")
        w = cid * NS + sid

        pltpu.sync_copy(src_hbm.at[pl.ds(w * EPW, EPW)], src_v)
        pltpu.sync_copy(dst_hbm.at[pl.ds(w * EPW, EPW)], dst_v)
        pltpu.sync_copy(ewp_hbm.at[pl.ds(w * EPW, EPW)], ewp_v)

        def zrow(i, _):
            for j in range(dw // 16):
                zb_v[i, pl.ds(j * 16, 16)] = jnp.zeros((16,), jnp.float32)
            return 0

        lax.fori_loop(0, 32, zrow, 0)
        for t in range(ROWS_PT // 32):
            pltpu.sync_copy(zb_v, accum.at[pl.ds(sid * ROWS_PT + t * 32, 32)])
        plsc.subcore_barrier()

        def scale(c, b):
            rib = rin.at[b]
            rob = rout.at[b]

            def ebody(e8, _):
                for u in range(8):
                    e = e8 * 8 + u
                    ewb = plsc.load_gather(
                        ewp_v, [jnp.full((16,), c * K + e, jnp.int32)])
                    for j in range(dw // 16):
                        sl = pl.ds(j * 16, 16)
                        rob[e, sl] = rib[e, sl] * ewb
                return 0

            lax.fori_loop(0, K // 8, ebody, 0)

        def gidx(c):
            return src_v.at[pl.ds(c * K, K)]

        for b in range(NB):
            pltpu.async_copy(table_hbm.at[gidx(b)], rin.at[b], gsem.at[b])

        def group(g, _):
            for b in range(NB):
                c = g * NB + b
                rib = rin.at[b]
                rob = rout.at[b]
                pltpu.make_async_copy(table_hbm.at[gidx(c)], rib,
                                      gsem.at[b]).wait()

                @pl.when(g > 0)
                def _():
                    pltpu.make_async_copy(rob, accum.at[didx_v[b]],
                                          ssem.at[b]).wait()

                for q in range(K // 16):
                    didx_v[b][pl.ds(q * 16, 16)] = dst_v[pl.ds(c * K + q * 16,
                                                               16)]
                scale(c, b)

                @pl.when(c + NB < NCHUNK)
                def _():
                    pltpu.async_copy(table_hbm.at[gidx(c + NB)], rib,
                                     gsem.at[b])

                pltpu.async_copy(rob, accum.at[didx_v[b]], ssem.at[b],
                                 add=True)
            return 0

        lax.fori_loop(0, NCHUNK // NB, group, 0)
        for b in range(NB):
            pltpu.make_async_copy(rout.at[b], accum.at[didx_v[b]],
                                  ssem.at[b]).wait()

        plsc.subcore_barrier()
        sl = pl.ds(sid * ROWS_PT, ROWS_PT)
        pltpu.sync_copy(accum.at[sl], out_hbm.at[cid, sl])

    return k(table, src1d, dst1d, ewp)


def _tc_mid(pa, pb, gamma, beta, alpha, W2):
    def body(pa_ref, pb_ref, g_ref, b_ref, a_ref, w2_ref, y2_ref, r1_ref):
        rst = jnp.concatenate(
            [pa_ref[0] + pa_ref[1], pb_ref[0] + pb_ref[1]], axis=1)
        m = jnp.mean(rst, axis=0, keepdims=True)
        q = jnp.mean(rst * rst, axis=0, keepdims=True)
        al = a_ref[...]
        out = rst - al * m
        var = q - (2.0 - al) * al * m * m
        h = _leaky(g_ref[...] * out * lax.rsqrt(var + EPS) + b_ref[...])
        r1_ref[...] = jnp.mean(h, axis=0, keepdims=True)
        y2_ref[...] = jnp.dot(h, w2_ref[...],
                              preferred_element_type=jnp.float32)

    return pl.pallas_call(
        body,
        out_shape=(
            jax.ShapeDtypeStruct((N, HID4), jnp.float32),
            jax.ShapeDtypeStruct((1, HID), jnp.float32),
        ),
    )(pa, pb, gamma, beta, alpha, W2)


def _tc_post(p, gamma, beta, alpha, r0, r1, We):
    def body(p_ref, g_ref, b_ref, a_ref, r0_ref, r1_ref, we_ref, out_ref):
        rst = p_ref[0, :N] + p_ref[1, :N]
        m = jnp.mean(rst, axis=0, keepdims=True)
        q = jnp.mean(rst * rst, axis=0, keepdims=True)
        al = a_ref[...]
        out = rst - al * m
        var = q - (2.0 - al) * al * m * m
        h = _leaky(g_ref[...] * out * lax.rsqrt(var + EPS) + b_ref[...])
        r2 = jnp.mean(h, axis=0, keepdims=True)
        emb = (jnp.dot(r0_ref[...], we_ref[0:D_IN, :],
                       preferred_element_type=jnp.float32)
               + jnp.dot(r1_ref[...], we_ref[D_IN:D_IN + HID, :],
                         preferred_element_type=jnp.float32)
               + jnp.dot(r2, we_ref[D_IN + HID:, :],
                         preferred_element_type=jnp.float32))
        em = jnp.mean(emb, axis=1, keepdims=True)
        ev = jnp.mean((emb - em) ** 2, axis=1, keepdims=True)
        out_ref[...] = _leaky((emb - em) * lax.rsqrt(ev + EPS))

    return pl.pallas_call(
        body,
        out_shape=jax.ShapeDtypeStruct((1, EMB), jnp.float32),
    )(p, gamma, beta, alpha, r0, r1, We)


def kernel(node_feats, edge_index, edge_weights, W1, W2, We,
           gamma1, beta1, alpha1, gamma2, beta2, alpha2):
    degp = _degrees(edge_index.reshape(2 * E))
    y1a, y1b, s, r0 = _tc_pre(node_feats, degp, W1)
    src1d = edge_index[0]
    dst1d = edge_index[1]
    ewp = _ewp_pass(src1d, dst1d, edge_weights, s)
    p1a, p1b = _edge_pass_dual(y1a, y1b, src1d, dst1d, ewp)
    y2, r1 = _tc_mid(p1a, p1b, gamma1.reshape(1, HID), beta1.reshape(1, HID),
                     alpha1.reshape(1, HID), W2)
    p2 = _edge_pass(y2, src1d, dst1d, ewp, HID4)
    out = _tc_post(p2, gamma2.reshape(1, HID4), beta2.reshape(1, HID4),
                   alpha2.reshape(1, HID4), r0, r1, We)
    return out


# speedup vs baseline: 1.7510x; 1.0449x over previous
import functools

import jax
import jax.numpy as jnp
from jax import lax
from jax.experimental import pallas as pl
from jax.experimental.pallas import tpu as pltpu
from jax.experimental.pallas import tpu_sc as plsc

N = 10000
E = 320000
D_IN = 128
HID = 128
HID4 = 32
EMB = 128
NEG_SLOPE = 0.01
EPS = 1e-5

NC = 2
NS = 16
NW = NC * NS
NPAD = 10240
EPW = E // NW
K = 16
NCHUNK = EPW // K
NB = 5
NB2 = 10
ROWS_PT = NPAD // NS


def _leaky(x):
    return jnp.where(x >= 0, x, NEG_SLOPE * x)


_IDXCH = 2000


def _degrees(edge_flat):
    mesh = plsc.VectorSubcoreMesh(core_axis_name="c", subcore_axis_name="s")

    @functools.partial(
        pl.kernel,
        mesh=mesh,
        out_type=jax.ShapeDtypeStruct((NW, NPAD), jnp.float32),
        scratch_types=[
            pltpu.VMEM((_IDXCH,), jnp.int32),
            pltpu.VMEM((NPAD,), jnp.float32),
        ],
        compiler_params=pltpu.CompilerParams(needs_layout_passes=False, use_tc_tiling_on_sc=False),
    )
    def k(edge_hbm, out_hbm, idx_v, hist_v):
        cid = lax.axis_index("c")
        sid = lax.axis_index("s")
        w = cid * NS + sid

        def zbody(i, _):
            hist_v[pl.ds(i * 16, 16)] = jnp.zeros((16,), jnp.float32)
            return 0

        lax.fori_loop(0, NPAD // 16, zbody, 0)

        per_sub = E // NS
        base = cid * E + sid * per_sub
        ones = jnp.full((16,), 1.0, jnp.float32)

        def chunk(cn, _):
            pltpu.sync_copy(edge_hbm.at[pl.ds(base + cn * _IDXCH, _IDXCH)],
                            idx_v)

            def ibody(t5, _):
                for u in range(5):
                    iv = idx_v[pl.ds((t5 * 5 + u) * 16, 16)]
                    plsc.addupdate_scatter(hist_v, [iv], ones)
                return 0

            lax.fori_loop(0, _IDXCH // 80, ibody, 0)
            return 0

        lax.fori_loop(0, per_sub // _IDXCH, chunk, 0)
        pltpu.sync_copy(hist_v, out_hbm.at[w])

    return k(edge_flat)


def _tc_pre(x, degp, W1):
    def body(x_ref, degp_ref, w1_ref, y1a_ref, y1b_ref, s_ref, r0_ref):
        xv = x_ref[...]
        y1 = jnp.dot(xv, w1_ref[...], preferred_element_type=jnp.float32)
        y1a_ref[...] = y1[:, :HID // 2]
        y1b_ref[...] = y1[:, HID // 2:]
        r0_ref[...] = jnp.mean(xv, axis=0, keepdims=True)
        dp = degp_ref[...]
        dout = jnp.sum(dp[:NS], axis=0, keepdims=True)
        din = jnp.sum(dp[NS:], axis=0, keepdims=True)
        deg = jnp.concatenate([dout, din], axis=0)
        s_ref[...] = lax.rsqrt(jnp.maximum(deg, 1.0))

    return pl.pallas_call(
        body,
        out_shape=(
            jax.ShapeDtypeStruct((N, HID // 2), jnp.float32),
            jax.ShapeDtypeStruct((N, HID // 2), jnp.float32),
            jax.ShapeDtypeStruct((2, NPAD), jnp.float32),
            jax.ShapeDtypeStruct((1, D_IN), jnp.float32),
        ),
    )(x, degp, W1)


def _ewp_pass(src1d, dst1d, ew, s):
    mesh = plsc.VectorSubcoreMesh(core_axis_name="c", subcore_axis_name="s")

    @functools.partial(
        pl.kernel,
        mesh=mesh,
        out_type=jax.ShapeDtypeStruct((E,), jnp.float32),
        scratch_types=[
            pltpu.VMEM((EPW,), jnp.int32),
            pltpu.VMEM((EPW,), jnp.int32),
            pltpu.VMEM((EPW,), jnp.float32),
            pltpu.VMEM((NPAD,), jnp.float32),
            pltpu.VMEM((NPAD,), jnp.float32),
        ],
        compiler_params=pltpu.CompilerParams(needs_layout_passes=False, use_tc_tiling_on_sc=False),
    )
    def k(src_hbm, dst_hbm, ew_hbm, s_hbm, out_hbm,
          src_v, dst_v, ewp_v, sout_v, sin_v):
        cid = lax.axis_index("c")
        sid = lax.axis_index("s")
        w = cid * NS + sid
        pltpu.sync_copy(src_hbm.at[pl.ds(w * EPW, EPW)], src_v)
        pltpu.sync_copy(dst_hbm.at[pl.ds(w * EPW, EPW)], dst_v)
        pltpu.sync_copy(ew_hbm.at[pl.ds(w * EPW, EPW)], ewp_v)
        pltpu.sync_copy(s_hbm.at[0], sout_v)
        pltpu.sync_copy(s_hbm.at[1], sin_v)

        def wbody(t5, _):
            for u in range(5):
                sl = pl.ds((t5 * 5 + u) * 16, 16)
                so = plsc.load_gather(sout_v, [src_v[sl]])
                si = plsc.load_gather(sin_v, [dst_v[sl]])
                ewp_v[sl] = ewp_v[sl] * so * si
            return 0

        lax.fori_loop(0, EPW // 80, wbody, 0)
        pltpu.sync_copy(ewp_v, out_hbm.at[pl.ds(w * EPW, EPW)])

    return k(src1d, dst1d, ew, s)


DW1 = HID // 2


def _edge_pass_dual(table_a, table_b, src1d, dst1d, ewp):
    mesh = plsc.VectorSubcoreMesh(core_axis_name="c", subcore_axis_name="s")

    @functools.partial(
        pl.kernel,
        mesh=mesh,
        out_type=(
            jax.ShapeDtypeStruct((NC, N, DW1), jnp.float32),
            jax.ShapeDtypeStruct((NC, N, DW1), jnp.float32),
        ),
        scratch_types=[
            pltpu.VMEM((EPW,), jnp.int32),
            pltpu.VMEM((EPW,), jnp.int32),
            [pltpu.VMEM((K,), jnp.int32)] * NB,
            pltpu.VMEM((EPW,), jnp.float32),
            pltpu.VMEM((NB, K, DW1), jnp.float32),
            pltpu.VMEM((NB, K, DW1), jnp.float32),
            pltpu.VMEM((NB, K, DW1), jnp.float32),
            pltpu.VMEM((NB, K, DW1), jnp.float32),
            pltpu.VMEM_SHARED((N, DW1), jnp.float32),
            pltpu.VMEM_SHARED((N, DW1), jnp.float32),
            pltpu.SemaphoreType.DMA((NB,)),
            pltpu.SemaphoreType.DMA((NB,)),
            pltpu.SemaphoreType.DMA((NB,)),
            pltpu.SemaphoreType.DMA((NB,)),
        ],
        compiler_params=pltpu.CompilerParams(needs_layout_passes=False,
                                             use_tc_tiling_on_sc=False),
    )
    def k(ta_hbm, tb_hbm, src_hbm, dst_hbm, ewp_hbm, outa_hbm, outb_hbm,
          src_v, dst_v, didx_v, ewp_v, rina, rinb, routa, routb,
          acca, accb, gsa, gsb, ssa, ssb):
        cid = lax.axis_index("c")
        sid = lax.axis_index("s")
        w = cid * NS + sid

        pltpu.sync_copy(src_hbm.at[pl.ds(w * EPW, EPW)], src_v)
        pltpu.sync_copy(dst_hbm.at[pl.ds(w * EPW, EPW)], dst_v)
        pltpu.sync_copy(ewp_hbm.at[pl.ds(w * EPW, EPW)], ewp_v)

        for i in range(K):
            for j in range(DW1 // 16):
                routa[0, i, pl.ds(j * 16, 16)] = jnp.zeros((16,), jnp.float32)

        @pl.when(sid < 10)
        def _():
            def zcopy(q, _):
                dst = pl.ds(sid * 1000 + q * 8, 8)
                pltpu.sync_copy(routa.at[0, pl.ds(0, 8)], acca.at[dst])
                pltpu.sync_copy(routa.at[0, pl.ds(0, 8)], accb.at[dst])
                return 0

            lax.fori_loop(0, 125, zcopy, 0)

        plsc.subcore_barrier()

        def scale(c, b):
            def ebody(e8, _):
                for u in range(8):
                    e = e8 * 8 + u
                    ewb = plsc.load_gather(
                        ewp_v, [jnp.full((16,), c * K + e, jnp.int32)])
                    for j in range(DW1 // 16):
                        sl = pl.ds(j * 16, 16)
                        routa[b, e, sl] = rina[b, e, sl] * ewb
                        routb[b, e, sl] = rinb[b, e, sl] * ewb
                return 0

            lax.fori_loop(0, K // 8, ebody, 0)

        def gidx(c):
            return src_v.at[pl.ds(c * K, K)]

        for b in range(NB):
            pltpu.async_copy(ta_hbm.at[gidx(b)], rina.at[b], gsa.at[b])
            pltpu.async_copy(tb_hbm.at[gidx(b)], rinb.at[b], gsb.at[b])

        def group(g, _):
            for b in range(NB):
                c = g * NB + b
                pltpu.make_async_copy(ta_hbm.at[gidx(c)], rina.at[b],
                                      gsa.at[b]).wait()
                pltpu.make_async_copy(tb_hbm.at[gidx(c)], rinb.at[b],
                                      gsb.at[b]).wait()

                @pl.when(g > 0)
                def _():
                    pltpu.make_async_copy(routa.at[b], acca.at[didx_v[b]],
                                          ssa.at[b]).wait()
                    pltpu.make_async_copy(routb.at[b], accb.at[didx_v[b]],
                                          ssb.at[b]).wait()

                for q in range(K // 16):
                    didx_v[b][pl.ds(q * 16, 16)] = dst_v[pl.ds(c * K + q * 16,
                                                               16)]
                scale(c, b)

                @pl.when(c + NB < NCHUNK)
                def _():
                    pltpu.async_copy(ta_hbm.at[gidx(c + NB)], rina.at[b],
                                     gsa.at[b])
                    pltpu.async_copy(tb_hbm.at[gidx(c + NB)], rinb.at[b],
                                     gsb.at[b])

                pltpu.async_copy(routa.at[b], acca.at[didx_v[b]], ssa.at[b],
                                 add=True)
                pltpu.async_copy(routb.at[b], accb.at[didx_v[b]], ssb.at[b],
                                 add=True)
            return 0

        lax.fori_loop(0, NCHUNK // NB, group, 0)
        for b in range(NB):
            pltpu.make_async_copy(routa.at[b], acca.at[didx_v[b]],
                                  ssa.at[b]).wait()
            pltpu.make_async_copy(routb.at[b], accb.at[didx_v[b]],
                                  ssb.at[b]).wait()

        plsc.subcore_barrier()

        @pl.when(sid < 10)
        def _():
            sl = pl.ds(sid * 1000, 1000)
            pltpu.sync_copy(acca.at[sl], outa_hbm.at[cid, sl])
            pltpu.sync_copy(accb.at[sl], outb_hbm.at[cid, sl])

    return k(table_a, table_b, src1d, dst1d, ewp)


def _edge_pass(table, src1d, dst1d, ewp, dw):
    mesh = plsc.VectorSubcoreMesh(core_axis_name="c", subcore_axis_name="s")

    @functools.partial(
        pl.kernel,
        mesh=mesh,
        out_type=jax.ShapeDtypeStruct((NC, NPAD, dw), jnp.float32),
        scratch_types=[
            pltpu.VMEM((EPW,), jnp.int32),
            pltpu.VMEM((EPW,), jnp.int32),
            [pltpu.VMEM((K,), jnp.int32)] * NB2,
            pltpu.VMEM((EPW,), jnp.float32),
            pltpu.VMEM((NB2, K, dw), jnp.float32),
            pltpu.VMEM((NB2, K, dw), jnp.float32),
            pltpu.VMEM((32, dw), jnp.float32),
            pltpu.VMEM_SHARED((NPAD, dw), jnp.float32),
            pltpu.SemaphoreType.DMA((NB2,)),
            pltpu.SemaphoreType.DMA((NB2,)),
        ],
        compiler_params=pltpu.CompilerParams(needs_layout_passes=False, use_tc_tiling_on_sc=False),
    )
    def k(table_hbm, src_hbm, dst_hbm, ewp_hbm, out_hbm,
          src_v, dst_v, didx_v, ewp_v, rin, rout, zb_v, accum,
          gsem, ssem):
        cid = lax.axis_index("c")
        sid = lax.axis_index("s")
        w = cid * NS + sid

        pltpu.sync_copy(src_hbm.at[pl.ds(w * EPW, EPW)], src_v)
        pltpu.sync_copy(dst_hbm.at[pl.ds(w * EPW, EPW)], dst_v)
        pltpu.sync_copy(ewp_hbm.at[pl.ds(w * EPW, EPW)], ewp_v)

        def zrow(i, _):
            for j in range(dw // 16):
                zb_v[i, pl.ds(j * 16, 16)] = jnp.zeros((16,), jnp.float32)
            return 0

        lax.fori_loop(0, 32, zrow, 0)
        for t in range(ROWS_PT // 32):
            pltpu.sync_copy(zb_v, accum.at[pl.ds(sid * ROWS_PT + t * 32, 32)])
        plsc.subcore_barrier()

        def scale(c, b):
            rib = rin.at[b]
            rob = rout.at[b]

            def ebody(e8, _):
                for u in range(8):
                    e = e8 * 8 + u
                    ewb = plsc.load_gather(
                        ewp_v, [jnp.full((16,), c * K + e, jnp.int32)])
                    for j in range(dw // 16):
                        sl = pl.ds(j * 16, 16)
                        rob[e, sl] = rib[e, sl] * ewb
                return 0

            lax.fori_loop(0, K // 8, ebody, 0)

        def gidx(c):
            return src_v.at[pl.ds(c * K, K)]

        for b in range(NB2):
            pltpu.async_copy(table_hbm.at[gidx(b)], rin.at[b], gsem.at[b])

        def fill_didx(c, b):
            for q in range(K // 16):
                didx_v[b][pl.ds(q * 16, 16)] = dst_v[pl.ds(c * K + q * 16,
                                                           16)]

        def group(g, _):
            for b in range(NB2):
                c = g * NB2 + b
                rib = rin.at[b]
                rob = rout.at[b]
                pltpu.make_async_copy(table_hbm.at[gidx(c)], rib,
                                      gsem.at[b]).wait()

                @pl.when(g > 0)
                def _():
                    pltpu.make_async_copy(rob, accum.at[didx_v[b]],
                                          ssem.at[b]).wait()

                fill_didx(c, b)
                scale(c, b)

                @pl.when(c + NB2 < NCHUNK)
                def _():
                    pltpu.async_copy(table_hbm.at[gidx(c + NB2)], rib,
                                     gsem.at[b])

                pltpu.async_copy(rob, accum.at[didx_v[b]], ssem.at[b],
                                 add=True)
            return 0

        ngroups = NCHUNK // NB2
        lax.fori_loop(0, ngroups, group, 0)
        for b in range(NCHUNK - ngroups * NB2):
            c = ngroups * NB2 + b
            rib = rin.at[b]
            rob = rout.at[b]
            pltpu.make_async_copy(table_hbm.at[gidx(c)], rib,
                                  gsem.at[b]).wait()
            pltpu.make_async_copy(rob, accum.at[didx_v[b]],
                                  ssem.at[b]).wait()
            fill_didx(c, b)
            scale(c, b)
            pltpu.async_copy(rob, accum.at[didx_v[b]], ssem.at[b],
                             add=True)
        for b in range(NB2):
            pltpu.make_async_copy(rout.at[b], accum.at[didx_v[b]],
                                  ssem.at[b]).wait()

        plsc.subcore_barrier()
        sl = pl.ds(sid * ROWS_PT, ROWS_PT)
        pltpu.sync_copy(accum.at[sl], out_hbm.at[cid, sl])

    return k(table, src1d, dst1d, ewp)


def _tc_mid(pa, pb, gamma, beta, alpha, W2):
    def body(pa_ref, pb_ref, g_ref, b_ref, a_ref, w2_ref, y2_ref, r1_ref):
        rst = jnp.concatenate(
            [pa_ref[0] + pa_ref[1], pb_ref[0] + pb_ref[1]], axis=1)
        m = jnp.mean(rst, axis=0, keepdims=True)
        q = jnp.mean(rst * rst, axis=0, keepdims=True)
        al = a_ref[...]
        out = rst - al * m
        var = q - (2.0 - al) * al * m * m
        h = _leaky(g_ref[...] * out * lax.rsqrt(var + EPS) + b_ref[...])
        r1_ref[...] = jnp.mean(h, axis=0, keepdims=True)
        y2_ref[...] = jnp.dot(h, w2_ref[...],
                              preferred_element_type=jnp.float32)

    return pl.pallas_call(
        body,
        out_shape=(
            jax.ShapeDtypeStruct((N, HID4), jnp.float32),
            jax.ShapeDtypeStruct((1, HID), jnp.float32),
        ),
    )(pa, pb, gamma, beta, alpha, W2)


def _tc_post(p, gamma, beta, alpha, r0, r1, We):
    def body(p_ref, g_ref, b_ref, a_ref, r0_ref, r1_ref, we_ref, out_ref):
        rst = p_ref[0, :N] + p_ref[1, :N]
        m = jnp.mean(rst, axis=0, keepdims=True)
        q = jnp.mean(rst * rst, axis=0, keepdims=True)
        al = a_ref[...]
        out = rst - al * m
        var = q - (2.0 - al) * al * m * m
        h = _leaky(g_ref[...] * out * lax.rsqrt(var + EPS) + b_ref[...])
        r2 = jnp.mean(h, axis=0, keepdims=True)
        emb = (jnp.dot(r0_ref[...], we_ref[0:D_IN, :],
                       preferred_element_type=jnp.float32)
               + jnp.dot(r1_ref[...], we_ref[D_IN:D_IN + HID, :],
                         preferred_element_type=jnp.float32)
               + jnp.dot(r2, we_ref[D_IN + HID:, :],
                         preferred_element_type=jnp.float32))
        em = jnp.mean(emb, axis=1, keepdims=True)
        ev = jnp.mean((emb - em) ** 2, axis=1, keepdims=True)
        out_ref[...] = _leaky((emb - em) * lax.rsqrt(ev + EPS))

    return pl.pallas_call(
        body,
        out_shape=jax.ShapeDtypeStruct((1, EMB), jnp.float32),
    )(p, gamma, beta, alpha, r0, r1, We)


def kernel(node_feats, edge_index, edge_weights, W1, W2, We,
           gamma1, beta1, alpha1, gamma2, beta2, alpha2):
    degp = _degrees(edge_index.reshape(2 * E))
    y1a, y1b, s, r0 = _tc_pre(node_feats, degp, W1)
    src1d = edge_index[0]
    dst1d = edge_index[1]
    ewp = _ewp_pass(src1d, dst1d, edge_weights, s)
    p1a, p1b = _edge_pass_dual(y1a, y1b, src1d, dst1d, ewp)
    y2, r1 = _tc_mid(p1a, p1b, gamma1.reshape(1, HID), beta1.reshape(1, HID),
                     alpha1.reshape(1, HID), W2)
    p2 = _edge_pass(y2, src1d, dst1d, ewp, HID4)
    out = _tc_post(p2, gamma2.reshape(1, HID4), beta2.reshape(1, HID4),
                   alpha2.reshape(1, HID4), r0, r1, We)
    return out


# speedup vs baseline: 1.7607x; 1.0055x over previous
import functools

import jax
import jax.numpy as jnp
from jax import lax
from jax.experimental import pallas as pl
from jax.experimental.pallas import tpu as pltpu
from jax.experimental.pallas import tpu_sc as plsc

N = 10000
E = 320000
D_IN = 128
HID = 128
HID4 = 32
EMB = 128
NEG_SLOPE = 0.01
EPS = 1e-5

NC = 2
NS = 16
NW = NC * NS
NPAD = 10240
EPW = E // NW
K = 16
NCHUNK = EPW // K
NB = 5
NB2 = 10
ROWS_PT = NPAD // NS


def _leaky(x):
    return jnp.where(x >= 0, x, NEG_SLOPE * x)


_IDXCH = 2000


def _degrees(edge_flat):
    mesh = plsc.VectorSubcoreMesh(core_axis_name="c", subcore_axis_name="s")

    @functools.partial(
        pl.kernel,
        mesh=mesh,
        out_type=jax.ShapeDtypeStruct((NW, NPAD), jnp.float32),
        scratch_types=[
            pltpu.VMEM((_IDXCH,), jnp.int32),
            pltpu.VMEM((NPAD,), jnp.float32),
        ],
        compiler_params=pltpu.CompilerParams(needs_layout_passes=False, use_tc_tiling_on_sc=False),
    )
    def k(edge_hbm, out_hbm, idx_v, hist_v):
        cid = lax.axis_index("c")
        sid = lax.axis_index("s")
        w = cid * NS + sid

        def zbody(i, _):
            hist_v[pl.ds(i * 16, 16)] = jnp.zeros((16,), jnp.float32)
            return 0

        lax.fori_loop(0, NPAD // 16, zbody, 0)

        per_sub = E // NS
        base = cid * E + sid * per_sub
        ones = jnp.full((16,), 1.0, jnp.float32)

        def chunk(cn, _):
            pltpu.sync_copy(edge_hbm.at[pl.ds(base + cn * _IDXCH, _IDXCH)],
                            idx_v)

            def ibody(t5, _):
                for u in range(5):
                    iv = idx_v[pl.ds((t5 * 5 + u) * 16, 16)]
                    plsc.addupdate_scatter(hist_v, [iv], ones)
                return 0

            lax.fori_loop(0, _IDXCH // 80, ibody, 0)
            return 0

        lax.fori_loop(0, per_sub // _IDXCH, chunk, 0)
        pltpu.sync_copy(hist_v, out_hbm.at[w])

    return k(edge_flat)


def _tc_y1(x, W1):
    def body(x_ref, w1_ref, y1a_ref, y1b_ref, r0_ref):
        xv = x_ref[...]
        y1 = jnp.dot(xv, w1_ref[...], preferred_element_type=jnp.float32)
        y1a_ref[...] = y1[:, :HID // 2]
        y1b_ref[...] = y1[:, HID // 2:]
        r0_ref[...] = jnp.mean(xv, axis=0, keepdims=True)

    return pl.pallas_call(
        body,
        out_shape=(
            jax.ShapeDtypeStruct((N, HID // 2), jnp.float32),
            jax.ShapeDtypeStruct((N, HID // 2), jnp.float32),
            jax.ShapeDtypeStruct((1, D_IN), jnp.float32),
        ),
    )(x, W1)


def _tc_s(degp):
    def body(degp_ref, s_ref):
        dp = degp_ref[...]
        dout = jnp.sum(dp[:NS], axis=0, keepdims=True)
        din = jnp.sum(dp[NS:], axis=0, keepdims=True)
        deg = jnp.concatenate([dout, din], axis=0)
        s_ref[...] = lax.rsqrt(jnp.maximum(deg, 1.0))

    return pl.pallas_call(
        body,
        out_shape=jax.ShapeDtypeStruct((2, NPAD), jnp.float32),
    )(degp)


def _ewp_pass(src1d, dst1d, ew, s):
    mesh = plsc.VectorSubcoreMesh(core_axis_name="c", subcore_axis_name="s")

    @functools.partial(
        pl.kernel,
        mesh=mesh,
        out_type=jax.ShapeDtypeStruct((E,), jnp.float32),
        scratch_types=[
            pltpu.VMEM((EPW,), jnp.int32),
            pltpu.VMEM((EPW,), jnp.int32),
            pltpu.VMEM((EPW,), jnp.float32),
            pltpu.VMEM((NPAD,), jnp.float32),
            pltpu.VMEM((NPAD,), jnp.float32),
        ],
        compiler_params=pltpu.CompilerParams(needs_layout_passes=False, use_tc_tiling_on_sc=False),
    )
    def k(src_hbm, dst_hbm, ew_hbm, s_hbm, out_hbm,
          src_v, dst_v, ewp_v, sout_v, sin_v):
        cid = lax.axis_index("c")
        sid = lax.axis_index("s")
        w = cid * NS + sid
        pltpu.sync_copy(src_hbm.at[pl.ds(w * EPW, EPW)], src_v)
        pltpu.sync_copy(dst_hbm.at[pl.ds(w * EPW, EPW)], dst_v)
        pltpu.sync_copy(ew_hbm.at[pl.ds(w * EPW, EPW)], ewp_v)
        pltpu.sync_copy(s_hbm.at[0], sout_v)
        pltpu.sync_copy(s_hbm.at[1], sin_v)

        def wbody(t5, _):
            for u in range(5):
                sl = pl.ds((t5 * 5 + u) * 16, 16)
                so = plsc.load_gather(sout_v, [src_v[sl]])
                si = plsc.load_gather(sin_v, [dst_v[sl]])
                ewp_v[sl] = ewp_v[sl] * so * si
            return 0

        lax.fori_loop(0, EPW // 80, wbody, 0)
        pltpu.sync_copy(ewp_v, out_hbm.at[pl.ds(w * EPW, EPW)])

    return k(src1d, dst1d, ew, s)


DW1 = HID // 2


def _edge_pass_dual(table_a, table_b, src1d, dst1d, ewp):
    mesh = plsc.VectorSubcoreMesh(core_axis_name="c", subcore_axis_name="s")

    @functools.partial(
        pl.kernel,
        mesh=mesh,
        out_type=(
            jax.ShapeDtypeStruct((NC, N, DW1), jnp.float32),
            jax.ShapeDtypeStruct((NC, N, DW1), jnp.float32),
        ),
        scratch_types=[
            pltpu.VMEM((EPW,), jnp.int32),
            pltpu.VMEM((EPW,), jnp.int32),
            [pltpu.VMEM((K,), jnp.int32)] * NB,
            pltpu.VMEM((EPW,), jnp.float32),
            pltpu.VMEM((NB, K, DW1), jnp.float32),
            pltpu.VMEM((NB, K, DW1), jnp.float32),
            pltpu.VMEM((NB, K, DW1), jnp.float32),
            pltpu.VMEM((NB, K, DW1), jnp.float32),
            pltpu.VMEM_SHARED((N, DW1), jnp.float32),
            pltpu.VMEM_SHARED((N, DW1), jnp.float32),
            pltpu.SemaphoreType.DMA((NB,)),
            pltpu.SemaphoreType.DMA((NB,)),
            pltpu.SemaphoreType.DMA((NB,)),
            pltpu.SemaphoreType.DMA((NB,)),
        ],
        compiler_params=pltpu.CompilerParams(needs_layout_passes=False,
                                             use_tc_tiling_on_sc=False),
    )
    def k(ta_hbm, tb_hbm, src_hbm, dst_hbm, ewp_hbm, outa_hbm, outb_hbm,
          src_v, dst_v, didx_v, ewp_v, rina, rinb, routa, routb,
          acca, accb, gsa, gsb, ssa, ssb):
        cid = lax.axis_index("c")
        sid = lax.axis_index("s")
        w = cid * NS + sid

        pltpu.sync_copy(src_hbm.at[pl.ds(w * EPW, EPW)], src_v)
        pltpu.sync_copy(dst_hbm.at[pl.ds(w * EPW, EPW)], dst_v)
        pltpu.sync_copy(ewp_hbm.at[pl.ds(w * EPW, EPW)], ewp_v)

        for i in range(K):
            for j in range(DW1 // 16):
                routa[0, i, pl.ds(j * 16, 16)] = jnp.zeros((16,), jnp.float32)

        @pl.when(sid < 10)
        def _():
            def zcopy(q, _):
                dst = pl.ds(sid * 1000 + q * 8, 8)
                pltpu.sync_copy(routa.at[0, pl.ds(0, 8)], acca.at[dst])
                pltpu.sync_copy(routa.at[0, pl.ds(0, 8)], accb.at[dst])
                return 0

            lax.fori_loop(0, 125, zcopy, 0)

        plsc.subcore_barrier()

        def scale(c, b):
            def ebody(e8, _):
                for u in range(8):
                    e = e8 * 8 + u
                    ewb = plsc.load_gather(
                        ewp_v, [jnp.full((16,), c * K + e, jnp.int32)])
                    for j in range(DW1 // 16):
                        sl = pl.ds(j * 16, 16)
                        routa[b, e, sl] = rina[b, e, sl] * ewb
                        routb[b, e, sl] = rinb[b, e, sl] * ewb
                return 0

            lax.fori_loop(0, K // 8, ebody, 0)

        def gidx(c):
            return src_v.at[pl.ds(c * K, K)]

        for b in range(NB):
            pltpu.async_copy(ta_hbm.at[gidx(b)], rina.at[b], gsa.at[b])
            pltpu.async_copy(tb_hbm.at[gidx(b)], rinb.at[b], gsb.at[b])

        def group(g, _):
            for b in range(NB):
                c = g * NB + b
                pltpu.make_async_copy(ta_hbm.at[gidx(c)], rina.at[b],
                                      gsa.at[b]).wait()
                pltpu.make_async_copy(tb_hbm.at[gidx(c)], rinb.at[b],
                                      gsb.at[b]).wait()

                @pl.when(g > 0)
                def _():
                    pltpu.make_async_copy(routa.at[b], acca.at[didx_v[b]],
                                          ssa.at[b]).wait()
                    pltpu.make_async_copy(routb.at[b], accb.at[didx_v[b]],
                                          ssb.at[b]).wait()

                for q in range(K // 16):
                    didx_v[b][pl.ds(q * 16, 16)] = dst_v[pl.ds(c * K + q * 16,
                                                               16)]
                scale(c, b)

                @pl.when(c + NB < NCHUNK)
                def _():
                    pltpu.async_copy(ta_hbm.at[gidx(c + NB)], rina.at[b],
                                     gsa.at[b])
                    pltpu.async_copy(tb_hbm.at[gidx(c + NB)], rinb.at[b],
                                     gsb.at[b])

                pltpu.async_copy(routa.at[b], acca.at[didx_v[b]], ssa.at[b],
                                 add=True)
                pltpu.async_copy(routb.at[b], accb.at[didx_v[b]], ssb.at[b],
                                 add=True)
            return 0

        lax.fori_loop(0, NCHUNK // NB, group, 0)
        for b in range(NB):
            pltpu.make_async_copy(routa.at[b], acca.at[didx_v[b]],
                                  ssa.at[b]).wait()
            pltpu.make_async_copy(routb.at[b], accb.at[didx_v[b]],
                                  ssb.at[b]).wait()

        plsc.subcore_barrier()

        @pl.when(sid < 10)
        def _():
            sl = pl.ds(sid * 1000, 1000)
            pltpu.sync_copy(acca.at[sl], outa_hbm.at[cid, sl])
            pltpu.sync_copy(accb.at[sl], outb_hbm.at[cid, sl])

    return k(table_a, table_b, src1d, dst1d, ewp)


def _edge_pass(table, src1d, dst1d, ewp, dw):
    mesh = plsc.VectorSubcoreMesh(core_axis_name="c", subcore_axis_name="s")

    @functools.partial(
        pl.kernel,
        mesh=mesh,
        out_type=jax.ShapeDtypeStruct((NC, NPAD, dw), jnp.float32),
        scratch_types=[
            pltpu.VMEM((EPW,), jnp.int32),
            pltpu.VMEM((EPW,), jnp.int32),
            [pltpu.VMEM((K,), jnp.int32)] * NB2,
            pltpu.VMEM((EPW,), jnp.float32),
            pltpu.VMEM((NB2, K, dw), jnp.float32),
            pltpu.VMEM((NB2, K, dw), jnp.float32),
            pltpu.VMEM((32, dw), jnp.float32),
            pltpu.VMEM_SHARED((NPAD, dw), jnp.float32),
            pltpu.SemaphoreType.DMA((NB2,)),
            pltpu.SemaphoreType.DMA((NB2,)),
        ],
        compiler_params=pltpu.CompilerParams(needs_layout_passes=False, use_tc_tiling_on_sc=False),
    )
    def k(table_hbm, src_hbm, dst_hbm, ewp_hbm, out_hbm,
          src_v, dst_v, didx_v, ewp_v, rin, rout, zb_v, accum,
          gsem, ssem):
        cid = lax.axis_index("c")
        sid = lax.axis_index("s")
        w = cid * NS + sid

        pltpu.sync_copy(src_hbm.at[pl.ds(w * EPW, EPW)], src_v)
        pltpu.sync_copy(dst_hbm.at[pl.ds(w * EPW, EPW)], dst_v)
        pltpu.sync_copy(ewp_hbm.at[pl.ds(w * EPW, EPW)], ewp_v)

        def zrow(i, _):
            for j in range(dw // 16):
                zb_v[i, pl.ds(j * 16, 16)] = jnp.zeros((16,), jnp.float32)
            return 0

        lax.fori_loop(0, 32, zrow, 0)
        for t in range(ROWS_PT // 32):
            pltpu.sync_copy(zb_v, accum.at[pl.ds(sid * ROWS_PT + t * 32, 32)])
        plsc.subcore_barrier()

        def scale(c, b):
            rib = rin.at[b]
            rob = rout.at[b]

            def ebody(e8, _):
                for u in range(8):
                    e = e8 * 8 + u
                    ewb = plsc.load_gather(
                        ewp_v, [jnp.full((16,), c * K + e, jnp.int32)])
                    for j in range(dw // 16):
                        sl = pl.ds(j * 16, 16)
                        rob[e, sl] = rib[e, sl] * ewb
                return 0

            lax.fori_loop(0, K // 8, ebody, 0)

        def gidx(c):
            return src_v.at[pl.ds(c * K, K)]

        for b in range(NB2):
            pltpu.async_copy(table_hbm.at[gidx(b)], rin.at[b], gsem.at[b])

        def fill_didx(c, b):
            for q in range(K // 16):
                didx_v[b][pl.ds(q * 16, 16)] = dst_v[pl.ds(c * K + q * 16,
                                                           16)]

        def group(g, _):
            for b in range(NB2):
                c = g * NB2 + b
                rib = rin.at[b]
                rob = rout.at[b]
                pltpu.make_async_copy(table_hbm.at[gidx(c)], rib,
                                      gsem.at[b]).wait()

                @pl.when(g > 0)
                def _():
                    pltpu.make_async_copy(rob, accum.at[didx_v[b]],
                                          ssem.at[b]).wait()

                fill_didx(c, b)
                scale(c, b)

                @pl.when(c + NB2 < NCHUNK)
                def _():
                    pltpu.async_copy(table_hbm.at[gidx(c + NB2)], rib,
                                     gsem.at[b])

                pltpu.async_copy(rob, accum.at[didx_v[b]], ssem.at[b],
                                 add=True)
            return 0

        ngroups = NCHUNK // NB2
        lax.fori_loop(0, ngroups, group, 0)
        for b in range(NCHUNK - ngroups * NB2):
            c = ngroups * NB2 + b
            rib = rin.at[b]
            rob = rout.at[b]
            pltpu.make_async_copy(table_hbm.at[gidx(c)], rib,
                                  gsem.at[b]).wait()
            pltpu.make_async_copy(rob, accum.at[didx_v[b]],
                                  ssem.at[b]).wait()
            fill_didx(c, b)
            scale(c, b)
            pltpu.async_copy(rob, accum.at[didx_v[b]], ssem.at[b],
                             add=True)
        for b in range(NB2):
            pltpu.make_async_copy(rout.at[b], accum.at[didx_v[b]],
                                  ssem.at[b]).wait()

        plsc.subcore_barrier()
        sl = pl.ds(sid * ROWS_PT, ROWS_PT)
        pltpu.sync_copy(accum.at[sl], out_hbm.at[cid, sl])

    return k(table, src1d, dst1d, ewp)


def _tc_mid(pa, pb, gamma, beta, alpha, W2):
    def body(pa_ref, pb_ref, g_ref, b_ref, a_ref, w2_ref, y2_ref, r1_ref):
        rst = jnp.concatenate(
            [pa_ref[0] + pa_ref[1], pb_ref[0] + pb_ref[1]], axis=1)
        m = jnp.mean(rst, axis=0, keepdims=True)
        q = jnp.mean(rst * rst, axis=0, keepdims=True)
        al = a_ref[...]
        out = rst - al * m
        var = q - (2.0 - al) * al * m * m
        h = _leaky(g_ref[...] * out * lax.rsqrt(var + EPS) + b_ref[...])
        r1_ref[...] = jnp.mean(h, axis=0, keepdims=True)
        y2_ref[...] = jnp.dot(h, w2_ref[...],
                              preferred_element_type=jnp.float32)

    return pl.pallas_call(
        body,
        out_shape=(
            jax.ShapeDtypeStruct((N, HID4), jnp.float32),
            jax.ShapeDtypeStruct((1, HID), jnp.float32),
        ),
    )(pa, pb, gamma, beta, alpha, W2)


def _tc_post(p, gamma, beta, alpha, r0, r1, We):
    def body(p_ref, g_ref, b_ref, a_ref, r0_ref, r1_ref, we_ref, out_ref):
        rst = p_ref[0, :N] + p_ref[1, :N]
        m = jnp.mean(rst, axis=0, keepdims=True)
        q = jnp.mean(rst * rst, axis=0, keepdims=True)
        al = a_ref[...]
        out = rst - al * m
        var = q - (2.0 - al) * al * m * m
        h = _leaky(g_ref[...] * out * lax.rsqrt(var + EPS) + b_ref[...])
        r2 = jnp.mean(h, axis=0, keepdims=True)
        emb = (jnp.dot(r0_ref[...], we_ref[0:D_IN, :],
                       preferred_element_type=jnp.float32)
               + jnp.dot(r1_ref[...], we_ref[D_IN:D_IN + HID, :],
                         preferred_element_type=jnp.float32)
               + jnp.dot(r2, we_ref[D_IN + HID:, :],
                         preferred_element_type=jnp.float32))
        em = jnp.mean(emb, axis=1, keepdims=True)
        ev = jnp.mean((emb - em) ** 2, axis=1, keepdims=True)
        out_ref[...] = _leaky((emb - em) * lax.rsqrt(ev + EPS))

    return pl.pallas_call(
        body,
        out_shape=jax.ShapeDtypeStruct((1, EMB), jnp.float32),
    )(p, gamma, beta, alpha, r0, r1, We)


def kernel(node_feats, edge_index, edge_weights, W1, W2, We,
           gamma1, beta1, alpha1, gamma2, beta2, alpha2):
    degp = _degrees(edge_index.reshape(2 * E))
    y1a, y1b, r0 = _tc_y1(node_feats, W1)
    s = _tc_s(degp)
    src1d = edge_index[0]
    dst1d = edge_index[1]
    ewp = _ewp_pass(src1d, dst1d, edge_weights, s)
    p1a, p1b = _edge_pass_dual(y1a, y1b, src1d, dst1d, ewp)
    y2, r1 = _tc_mid(p1a, p1b, gamma1.reshape(1, HID), beta1.reshape(1, HID),
                     alpha1.reshape(1, HID), W2)
    p2 = _edge_pass(y2, src1d, dst1d, ewp, HID4)
    out = _tc_post(p2, gamma2.reshape(1, HID4), beta2.reshape(1, HID4),
                   alpha2.reshape(1, HID4), r0, r1, We)
    return out
